# baseline (speedup 1.0000x reference)
import functools

import jax
import jax.numpy as jnp
from jax import lax
from jax.experimental import pallas as pl
from jax.experimental.pallas import tpu as pltpu
from jax.experimental.pallas import tpu_sc as plsc

D_MODEL = 64
MAX_LEN = 512
NUM_CORES = 2
NUM_SUBCORES = 16
NUM_WORKERS = NUM_CORES * NUM_SUBCORES

CHUNK = 256
IDX_W = 128
N_STREAMS = CHUNK // IDX_W


def _pos_encoding():
    pos = jnp.arange(MAX_LEN, dtype=jnp.float32)[:, None]
    _2i = jnp.arange(0, D_MODEL, 2, dtype=jnp.float32)
    ang = pos / jnp.power(10000.0, _2i / D_MODEL)
    pe = jnp.zeros((MAX_LEN, D_MODEL), dtype=jnp.float32)
    pe = pe.at[:, 0::2].set(jnp.sin(ang))
    pe = pe.at[:, 1::2].set(jnp.cos(ang))
    return pe


@jax.jit
def _embed(x, weight, pe):
    batch, seq = x.shape
    n_tokens = batch * seq
    sub = seq // CHUNK
    chunks = n_tokens // (NUM_WORKERS * CHUNK)
    mesh = plsc.VectorSubcoreMesh(core_axis_name="c", subcore_axis_name="s")

    @functools.partial(
        pl.kernel,
        out_type=jax.ShapeDtypeStruct((n_tokens // 2, 128), jnp.float32),
        mesh=mesh,
        compiler_params=pltpu.CompilerParams(use_tc_tiling_on_sc=False),
        scratch_types=[
            pltpu.VMEM((MAX_LEN, D_MODEL), jnp.float32),
            pltpu.VMEM((2, CHUNK), jnp.int32),
            pltpu.VMEM((2, CHUNK, D_MODEL), jnp.float32),
            pltpu.VMEM((2, CHUNK // 2, 128), jnp.float32),
            pltpu.SemaphoreType.DMA,
            pltpu.SemaphoreType.DMA,
            pltpu.SemaphoreType.DMA,
            pltpu.SemaphoreType.DMA,
            pltpu.SemaphoreType.DMA,
            pltpu.SemaphoreType.DMA,
        ],
    )
    def kern(x_hbm, w_hbm, pe_hbm, out_hbm, pe_v, idx_v, rows_v, stg_v,
             sg0, sg1, si0, si1, so0, so1):
        sem_g = (sg0, sg1)
        sem_i = (si0, si1)
        sem_o = (so0, so1)
        wid = lax.axis_index("s") * NUM_CORES + lax.axis_index("c")
        pltpu.sync_copy(pe_hbm, pe_v)
        c0 = wid * chunks

        def load_idx(ci, buf, sem):
            gc = c0 + ci
            return pltpu.async_copy(
                x_hbm.at[gc // sub].at[pl.ds((gc % sub) * CHUNK, CHUNK)],
                idx_v.at[buf],
                sem,
            )

        def fire_gathers(buf, sem):
            for j in range(N_STREAMS):
                pltpu.async_copy(
                    w_hbm.at[idx_v.at[buf].at[pl.ds(j * IDX_W, IDX_W)]],
                    rows_v.at[buf].at[pl.ds(j * IDX_W, IDX_W)],
                    sem,
                )

        def drain_gathers(buf, sem):
            for j in range(N_STREAMS):
                pltpu.make_async_copy(
                    w_hbm.at[idx_v.at[buf].at[pl.ds(j * IDX_W, IDX_W)]],
                    rows_v.at[buf].at[pl.ds(j * IDX_W, IDX_W)],
                    sem,
                ).wait()

        def out_slice(ci):
            return out_hbm.at[pl.ds((c0 + ci) * (CHUNK // 2), CHUNK // 2)]

        load_idx(0, 0, sem_i[0]).wait()
        fire_gathers(0, sem_g[0])
        load_idx(1, 1, sem_i[1])

        @pl.loop(0, chunks, step=2)
        def _(c):
            for b in range(2):
                cc = c + b
                o = 1 - b
                drain_gathers(b, sem_g[b])

                @pl.when(cc + 2 < chunks)
                def _():
                    load_idx(cc + 2, b, sem_i[b])

                @pl.when(cc > 0)
                def _():
                    pltpu.make_async_copy(
                        stg_v.at[o], out_slice(cc - 1), sem_o[o],
                    ).wait()

                @pl.when(cc + 1 < chunks)
                def _():
                    gc1 = c0 + cc + 1
                    pltpu.make_async_copy(
                        x_hbm.at[gc1 // sub].at[pl.ds((gc1 % sub) * CHUNK,
                                                      CHUNK)],
                        idx_v.at[o],
                        sem_i[o],
                    ).wait()
                    fire_gathers(o, sem_g[o])

                s0 = ((c0 + cc) % sub) * CHUNK

                @pl.loop(0, CHUNK // 2)
                def _(rr):
                    for t in range(2):
                        for c4 in range(D_MODEL // 16):
                            sl = pl.ds(c4 * 16, 16)
                            stg_v[b, rr, pl.ds(t * 64 + c4 * 16, 16)] = (
                                rows_v[b, 2 * rr + t, sl]
                                + pe_v[s0 + 2 * rr + t, sl]
                            )

                pltpu.async_copy(stg_v.at[b], out_slice(cc), sem_o[b])

        pltpu.make_async_copy(
            stg_v.at[(chunks - 1) % 2],
            out_slice(chunks - 1),
            sem_o[(chunks - 1) % 2],
        ).wait()

    return kern(x, weight, pe)


def kernel(x, weight):
    b, l = x.shape
    return _embed(x, weight, _pos_encoding()).reshape(b, l, D_MODEL)
